# baseline (speedup 1.0000x reference)
import jax
import jax.numpy as jnp
from jax.experimental import pallas as pl
from jax.experimental.pallas import tpu as pltpu

_GRID = 16


def _copy_block(t_in, id_in, t_out, id_out):
    t_out[...] = t_in[...]

    @pl.when(pl.program_id(0) == 0)
    def _():
        id_out[...] = id_in[...]


def kernel(t, id=None):
    if id is None:
        id = jnp.tile(jnp.array([[0]], dtype=jnp.int32), (t.shape[0], 1))
    id_sq = id.reshape(128, id.size // 128)
    blk = t.shape[0] // _GRID
    t_out, id_out = pl.pallas_call(
        _copy_block,
        grid=(_GRID,),
        out_shape=(
            jax.ShapeDtypeStruct(t.shape, t.dtype),
            jax.ShapeDtypeStruct(id_sq.shape, id_sq.dtype),
        ),
        in_specs=[
            pl.BlockSpec((blk, t.shape[1]), lambda i: (i, 0)),
            pl.BlockSpec(id_sq.shape, lambda i: (0, 0)),
        ],
        out_specs=(
            pl.BlockSpec((blk, t.shape[1]), lambda i: (i, 0)),
            pl.BlockSpec(id_sq.shape, lambda i: (0, 0)),
        ),
        compiler_params=pltpu.CompilerParams(
            dimension_semantics=("arbitrary",),
        ),
    )(t, id_sq)
    return t_out, id_out.reshape(id.shape)


# speedup vs baseline: 1.2558x; 1.2558x over previous
import jax
import jax.numpy as jnp
from jax.experimental import pallas as pl
from jax.experimental.pallas import tpu as pltpu

_GRID = 4


def _copy_block(t_in, id_in, t_out, id_out):
    t_out[...] = t_in[...]

    @pl.when(pl.program_id(0) == 0)
    def _():
        id_out[...] = id_in[...]


def kernel(t, id=None):
    if id is None:
        id = jnp.tile(jnp.array([[0]], dtype=jnp.int32), (t.shape[0], 1))
    id_sq = id.reshape(128, id.size // 128)
    blk = t.shape[0] // _GRID
    t_out, id_out = pl.pallas_call(
        _copy_block,
        grid=(_GRID,),
        out_shape=(
            jax.ShapeDtypeStruct(t.shape, t.dtype),
            jax.ShapeDtypeStruct(id_sq.shape, id_sq.dtype),
        ),
        in_specs=[
            pl.BlockSpec((blk, t.shape[1]), lambda i: (i, 0)),
            pl.BlockSpec(id_sq.shape, lambda i: (0, 0)),
        ],
        out_specs=(
            pl.BlockSpec((blk, t.shape[1]), lambda i: (i, 0)),
            pl.BlockSpec(id_sq.shape, lambda i: (0, 0)),
        ),
        compiler_params=pltpu.CompilerParams(
            dimension_semantics=("arbitrary",),
        ),
    )(t, id_sq)
    return t_out, id_out.reshape(id.shape)
